# baseline (device time: 62653 ns/iter reference)
import os

import jax
import jax.numpy as jnp
from jax import lax
from jax.experimental import pallas as pl
from jax.experimental.pallas import tpu as pltpu

_VARIANT = os.environ.get("KERNEL_VARIANT", "full")

N_DEV = 8
SQ = 1024
SKV = 1024
H_PER = 8
DH = 128
DM = 1024
BLK = 64
SCALE = 0.08838834764831843

PARTS = [(0, 384, 384), (384, 384, 768), (768, 256, 1024)]


def kernel(x, Wq, K_ext, V_ext, Wo):
    xb = x[0]
    k3 = K_ext[0]
    v3 = V_ext[0]

    def body(x_ref, k_ref, v_ref, wq_hbm, wo_hbm, out_ref,
             wq_vmem, wo_vmem, kst, vst, work0, work1, work2,
             st00, st01, st02, st10, st11, st12, st20, st21, st22,
             w_sems, kv_sems, rs_send_sems, rs_recv_sems,
             ag_send_sems, ag_recv_sems):
        my_pos = lax.axis_index("i")
        work = [work0, work1, work2]
        stage = [[st00, st01, st02], [st10, st11, st12], [st20, st21, st22]]

        wq_dma = pltpu.make_async_copy(
            wq_hbm.at[:, pl.ds(my_pos * (H_PER * DH), H_PER * DH)],
            wq_vmem, w_sems.at[0])
        wo_dma = pltpu.make_async_copy(
            wo_hbm.at[pl.ds(my_pos * (H_PER * DH), H_PER * DH), :],
            wo_vmem, w_sems.at[1])
        wq_dma.start()
        wo_dma.start()
        kv_dmas = []
        for h in range(H_PER):
            kd = pltpu.make_async_copy(
                k_ref.at[:, h, :], kst.at[h], kv_sems.at[0, h])
            vd = pltpu.make_async_copy(
                v_ref.at[:, h, :], vst.at[h], kv_sems.at[1, h])
            kd.start()
            vd.start()
            kv_dmas.append((kd, vd))

        partners = [
            my_pos ^ 1,
            (my_pos & 4) | ((my_pos & 3) ^ 3),
            my_pos ^ 4,
        ]
        bits = [
            (my_pos ^ (my_pos >> 1)) & 1,
            (my_pos >> 1) & 1,
            (my_pos >> 2) & 1,
        ]

        barrier_sem = pltpu.get_barrier_semaphore()
        for p in partners:
            pl.semaphore_signal(
                barrier_sem, inc=1,
                device_id=(p,), device_id_type=pl.DeviceIdType.MESH,
            )
        pl.semaphore_wait(barrier_sem, 3)

        wq_dma.wait()
        wo_dma.wait()
        wq_bf = (wq_vmem[:, :] * SCALE).astype(jnp.bfloat16)
        wo_bf = wo_vmem[:, :].astype(jnp.bfloat16)

        def compute_part(t):
            row0, nr, kvc = PARTS[t]
            xq = x_ref[row0:row0 + nr, :].astype(jnp.bfloat16)
            q = jnp.dot(
                xq, wq_bf, preferred_element_type=jnp.float32
            ).astype(jnp.bfloat16)
            qb = (row0 + lax.broadcasted_iota(jnp.int32, (nr, kvc), 0)) // BLK
            kb = lax.broadcasted_iota(jnp.int32, (nr, kvc), 1) // BLK
            mask = kb <= qb
            ctx_heads = []
            for h in range(H_PER):
                if t == 0:
                    kv_dmas[h][0].wait()
                    kv_dmas[h][1].wait()
                sl = slice(h * DH, (h + 1) * DH)
                k_h = kst[h, 0:kvc, :].astype(jnp.bfloat16)
                s = lax.dot_general(
                    q[:, sl], k_h, (((1,), (1,)), ((), ())),
                    preferred_element_type=jnp.float32,
                )
                e = jnp.where(mask, jnp.exp(s), 0.0)
                w = (e / jnp.sum(e, axis=1, keepdims=True)).astype(jnp.bfloat16)
                ctx_heads.append(jnp.dot(
                    w, vst[h, 0:kvc, :].astype(jnp.bfloat16),
                    preferred_element_type=jnp.float32,
                ).astype(jnp.bfloat16))
            ctx = jnp.concatenate(ctx_heads, axis=1)
            acc = jnp.dot(ctx, wo_bf, preferred_element_type=jnp.float32)
            work[t][:, :] = acc.astype(jnp.bfloat16)

        state = {t: (jnp.int32(0), PARTS[t][1]) for t in range(3)}
        rs_pending = {}
        ag_pending = {}

        def rs_send(t, r):
            d = (t + r) % 3
            start, sz = state[t]
            half = sz // 2
            b = bits[d]
            keep_start = start + b * half
            send_start = start + (1 - b) * half
            rdma = pltpu.make_async_remote_copy(
                src_ref=work[t].at[pl.ds(send_start, half), :],
                dst_ref=stage[t][r],
                send_sem=rs_send_sems.at[t, r],
                recv_sem=rs_recv_sems.at[t, r],
                device_id=(partners[d],),
                device_id_type=pl.DeviceIdType.MESH,
            )
            rdma.start()
            rs_pending[(t, r)] = (rdma, keep_start, half)
            state[t] = (keep_start, half)

        def rs_wait(t, r):
            rdma, keep_start, half = rs_pending[(t, r)]
            rdma.wait()
            work[t][pl.ds(keep_start, half), :] = (
                work[t][pl.ds(keep_start, half), :] + stage[t][r][:, :]
            )

        def ag_send(t, r):
            d = (t + 2 - r) % 3
            start, sz = state[t]
            rdma = pltpu.make_async_remote_copy(
                src_ref=work[t].at[pl.ds(start, sz), :],
                dst_ref=work[t].at[pl.ds(start, sz), :],
                send_sem=ag_send_sems.at[t, r],
                recv_sem=ag_recv_sems.at[t, r],
                device_id=(partners[d],),
                device_id_type=pl.DeviceIdType.MESH,
            )
            rdma.start()
            ag_pending[(t, r)] = rdma
            state[t] = (start - bits[d] * sz, 2 * sz)

        def ag_wait(t, r):
            ag_pending[(t, r)].wait()

        def emit_out(t):
            row0, nr, _ = PARTS[t]
            out_ref[0, row0:row0 + nr, :] = work[t][:, :].astype(jnp.float32)

        if _VARIANT == "nocomm":
            for t in range(3):
                compute_part(t)
            emit_out(0); emit_out(1); emit_out(2)
            return

        compute_part(0)
        rs_send(0, 0)
        compute_part(1)
        rs_send(1, 0); rs_wait(0, 0); rs_send(0, 1)
        compute_part(2)
        rs_send(2, 0); rs_wait(1, 0); rs_send(1, 1)
        rs_wait(0, 1); rs_send(0, 2)
        rs_wait(2, 0); rs_send(2, 1); rs_wait(1, 1); rs_send(1, 2)
        rs_wait(0, 2); ag_send(0, 0)
        rs_wait(2, 1); rs_send(2, 2); rs_wait(1, 2); ag_send(1, 0)
        ag_wait(0, 0); ag_send(0, 1)
        rs_wait(2, 2); ag_send(2, 0); ag_wait(1, 0); ag_send(1, 1)
        ag_wait(0, 1); ag_send(0, 2)
        ag_wait(2, 0); ag_send(2, 1); ag_wait(1, 1); ag_send(1, 2)
        ag_wait(0, 2); emit_out(0)
        ag_wait(2, 1); ag_send(2, 2); ag_wait(1, 2); emit_out(1)
        ag_wait(2, 2); emit_out(2)

    out = pl.pallas_call(
        body,
        out_shape=jax.ShapeDtypeStruct((1, SQ, DM), jnp.float32),
        in_specs=[
            pl.BlockSpec(memory_space=pltpu.VMEM),
            pl.BlockSpec(memory_space=pltpu.VMEM),
            pl.BlockSpec(memory_space=pltpu.VMEM),
            pl.BlockSpec(memory_space=pl.ANY),
            pl.BlockSpec(memory_space=pl.ANY),
        ],
        out_specs=pl.BlockSpec(memory_space=pltpu.VMEM),
        scratch_shapes=[
            pltpu.VMEM((DM, H_PER * DH), jnp.float32),
            pltpu.VMEM((H_PER * DH, DM), jnp.float32),
            pltpu.VMEM((H_PER, SKV, DH), jnp.float32),
            pltpu.VMEM((H_PER, SKV, DH), jnp.float32),
            pltpu.VMEM((384, DM), jnp.bfloat16),
            pltpu.VMEM((384, DM), jnp.bfloat16),
            pltpu.VMEM((256, DM), jnp.bfloat16),
            pltpu.VMEM((192, DM), jnp.bfloat16),
            pltpu.VMEM((96, DM), jnp.bfloat16),
            pltpu.VMEM((48, DM), jnp.bfloat16),
            pltpu.VMEM((192, DM), jnp.bfloat16),
            pltpu.VMEM((96, DM), jnp.bfloat16),
            pltpu.VMEM((48, DM), jnp.bfloat16),
            pltpu.VMEM((128, DM), jnp.bfloat16),
            pltpu.VMEM((64, DM), jnp.bfloat16),
            pltpu.VMEM((32, DM), jnp.bfloat16),
            pltpu.SemaphoreType.DMA((2,)),
            pltpu.SemaphoreType.DMA((2, H_PER)),
            pltpu.SemaphoreType.DMA((3, 3)),
            pltpu.SemaphoreType.DMA((3, 3)),
            pltpu.SemaphoreType.DMA((3, 3)),
            pltpu.SemaphoreType.DMA((3, 3)),
        ],
        compiler_params=pltpu.CompilerParams(collective_id=0),
    )(xb, k3, v3, Wq, Wo)
    return out


# device time: 61272 ns/iter; 1.0225x vs baseline; 1.0225x over previous
import os

import jax
import jax.numpy as jnp
from jax import lax
from jax.experimental import pallas as pl
from jax.experimental.pallas import tpu as pltpu

_VARIANT = os.environ.get("KERNEL_VARIANT", "full")

N_DEV = 8
SQ = 1024
SKV = 1024
H_PER = 8
DH = 128
DM = 1024
BLK = 64
SCALE = 0.08838834764831843

PARTS = [(0, 384, 384), (384, 384, 768), (768, 256, 1024)]


def kernel(x, Wq, K_ext, V_ext, Wo):
    xb = x[0]
    k3 = K_ext[0]
    v3 = V_ext[0]

    def body(x_ref, k_ref, v_ref, wq_hbm, wo_hbm, out_ref,
             wq_vmem, wo_vmem, kst, vst, work0, work1, work2,
             st00, st01, st02, st10, st11, st12, st20, st21, st22,
             w_sems, kv_sems, rs_send_sems, rs_recv_sems,
             ag_send_sems, ag_recv_sems):
        my_pos = lax.axis_index("i")
        work = [work0, work1, work2]
        stage = [[st00, st01, st02], [st10, st11, st12], [st20, st21, st22]]

        wq_dma = pltpu.make_async_copy(
            wq_hbm.at[:, pl.ds(my_pos * (H_PER * DH), H_PER * DH)],
            wq_vmem, w_sems.at[0])
        wo_dma = pltpu.make_async_copy(
            wo_hbm.at[pl.ds(my_pos * (H_PER * DH), H_PER * DH), :],
            wo_vmem, w_sems.at[1])
        wq_dma.start()
        wo_dma.start()
        kv_dmas = []
        for h in range(H_PER):
            kd = pltpu.make_async_copy(
                k_ref.at[:, h, :], kst.at[h], kv_sems.at[0, h])
            vd = pltpu.make_async_copy(
                v_ref.at[:, h, :], vst.at[h], kv_sems.at[1, h])
            kd.start()
            vd.start()
            kv_dmas.append((kd, vd))

        partners = [
            my_pos ^ 1,
            (my_pos & 4) | ((my_pos & 3) ^ 3),
            my_pos ^ 4,
        ]
        bits = [
            (my_pos ^ (my_pos >> 1)) & 1,
            (my_pos >> 1) & 1,
            (my_pos >> 2) & 1,
        ]

        barrier_sem = pltpu.get_barrier_semaphore()
        for p in partners:
            pl.semaphore_signal(
                barrier_sem, inc=1,
                device_id=(p,), device_id_type=pl.DeviceIdType.MESH,
            )
        pl.semaphore_wait(barrier_sem, 3)

        wq_dma.wait()
        wo_dma.wait()
        for kd, vd in kv_dmas:
            kd.wait()
            vd.wait()
        wq_bf = (wq_vmem[:, :] * SCALE).astype(jnp.bfloat16)
        wo_bf = wo_vmem[:, :].astype(jnp.bfloat16)

        def compute_part(t):
            row0, nr, kvc = PARTS[t]
            xq = x_ref[row0:row0 + nr, :].astype(jnp.bfloat16)
            q = jnp.dot(
                xq, wq_bf, preferred_element_type=jnp.float32
            ).astype(jnp.bfloat16)
            qb = (row0 + lax.broadcasted_iota(jnp.int32, (nr, kvc), 0)) // BLK
            kb = lax.broadcasted_iota(jnp.int32, (nr, kvc), 1) // BLK
            mask = kb <= qb
            ctx_heads = []
            for h in range(H_PER):
                sl = slice(h * DH, (h + 1) * DH)
                k_h = kst[h, 0:kvc, :].astype(jnp.bfloat16)
                s = lax.dot_general(
                    q[:, sl], k_h, (((1,), (1,)), ((), ())),
                    preferred_element_type=jnp.float32,
                )
                e = jnp.where(mask, jnp.exp(s), 0.0)
                w = (e / jnp.sum(e, axis=1, keepdims=True)).astype(jnp.bfloat16)
                ctx_heads.append(jnp.dot(
                    w, vst[h, 0:kvc, :].astype(jnp.bfloat16),
                    preferred_element_type=jnp.float32,
                ).astype(jnp.bfloat16))
            ctx = jnp.concatenate(ctx_heads, axis=1)
            acc = jnp.dot(ctx, wo_bf, preferred_element_type=jnp.float32)
            work[t][:, :] = acc.astype(jnp.bfloat16)

        state = {t: (jnp.int32(0), PARTS[t][1]) for t in range(3)}
        rs_pending = {}
        ag_pending = {}

        def rs_send(t, r):
            d = (t + r) % 3
            start, sz = state[t]
            half = sz // 2
            b = bits[d]
            keep_start = start + b * half
            send_start = start + (1 - b) * half
            rdma = pltpu.make_async_remote_copy(
                src_ref=work[t].at[pl.ds(send_start, half), :],
                dst_ref=stage[t][r],
                send_sem=rs_send_sems.at[t, r],
                recv_sem=rs_recv_sems.at[t, r],
                device_id=(partners[d],),
                device_id_type=pl.DeviceIdType.MESH,
            )
            rdma.start()
            rs_pending[(t, r)] = (rdma, keep_start, half)
            state[t] = (keep_start, half)

        def rs_wait(t, r):
            rdma, keep_start, half = rs_pending[(t, r)]
            rdma.wait()
            work[t][pl.ds(keep_start, half), :] = (
                work[t][pl.ds(keep_start, half), :] + stage[t][r][:, :]
            )

        def ag_send(t, r):
            d = (t + 2 - r) % 3
            start, sz = state[t]
            rdma = pltpu.make_async_remote_copy(
                src_ref=work[t].at[pl.ds(start, sz), :],
                dst_ref=work[t].at[pl.ds(start, sz), :],
                send_sem=ag_send_sems.at[t, r],
                recv_sem=ag_recv_sems.at[t, r],
                device_id=(partners[d],),
                device_id_type=pl.DeviceIdType.MESH,
            )
            rdma.start()
            ag_pending[(t, r)] = rdma
            state[t] = (start - bits[d] * sz, 2 * sz)

        def ag_wait(t, r):
            ag_pending[(t, r)].wait()

        def emit_out(t):
            row0, nr, _ = PARTS[t]
            out_ref[0, row0:row0 + nr, :] = work[t][:, :].astype(jnp.float32)

        if _VARIANT == "nocomm":
            for t in range(3):
                compute_part(t)
            emit_out(0); emit_out(1); emit_out(2)
            return

        compute_part(0)
        rs_send(0, 0)
        compute_part(1)
        rs_send(1, 0); rs_wait(0, 0); rs_send(0, 1)
        compute_part(2)
        rs_send(2, 0); rs_wait(1, 0); rs_send(1, 1)
        rs_wait(0, 1); rs_send(0, 2)
        rs_wait(2, 0); rs_send(2, 1); rs_wait(1, 1); rs_send(1, 2)
        rs_wait(0, 2); ag_send(0, 0)
        rs_wait(2, 1); rs_send(2, 2); rs_wait(1, 2); ag_send(1, 0)
        ag_wait(0, 0); ag_send(0, 1)
        rs_wait(2, 2); ag_send(2, 0); ag_wait(1, 0); ag_send(1, 1)
        ag_wait(0, 1); ag_send(0, 2)
        ag_wait(2, 0); ag_send(2, 1); ag_wait(1, 1); ag_send(1, 2)
        ag_wait(0, 2); emit_out(0)
        ag_wait(2, 1); ag_send(2, 2); ag_wait(1, 2); emit_out(1)
        ag_wait(2, 2); emit_out(2)

    out = pl.pallas_call(
        body,
        out_shape=jax.ShapeDtypeStruct((1, SQ, DM), jnp.float32),
        in_specs=[
            pl.BlockSpec(memory_space=pltpu.VMEM),
            pl.BlockSpec(memory_space=pltpu.VMEM),
            pl.BlockSpec(memory_space=pltpu.VMEM),
            pl.BlockSpec(memory_space=pl.ANY),
            pl.BlockSpec(memory_space=pl.ANY),
        ],
        out_specs=pl.BlockSpec(memory_space=pltpu.VMEM),
        scratch_shapes=[
            pltpu.VMEM((DM, H_PER * DH), jnp.float32),
            pltpu.VMEM((H_PER * DH, DM), jnp.float32),
            pltpu.VMEM((H_PER, SKV, DH), jnp.float32),
            pltpu.VMEM((H_PER, SKV, DH), jnp.float32),
            pltpu.VMEM((384, DM), jnp.bfloat16),
            pltpu.VMEM((384, DM), jnp.bfloat16),
            pltpu.VMEM((256, DM), jnp.bfloat16),
            pltpu.VMEM((192, DM), jnp.bfloat16),
            pltpu.VMEM((96, DM), jnp.bfloat16),
            pltpu.VMEM((48, DM), jnp.bfloat16),
            pltpu.VMEM((192, DM), jnp.bfloat16),
            pltpu.VMEM((96, DM), jnp.bfloat16),
            pltpu.VMEM((48, DM), jnp.bfloat16),
            pltpu.VMEM((128, DM), jnp.bfloat16),
            pltpu.VMEM((64, DM), jnp.bfloat16),
            pltpu.VMEM((32, DM), jnp.bfloat16),
            pltpu.SemaphoreType.DMA((2,)),
            pltpu.SemaphoreType.DMA((2, H_PER)),
            pltpu.SemaphoreType.DMA((3, 3)),
            pltpu.SemaphoreType.DMA((3, 3)),
            pltpu.SemaphoreType.DMA((3, 3)),
            pltpu.SemaphoreType.DMA((3, 3)),
        ],
        compiler_params=pltpu.CompilerParams(collective_id=0),
    )(xb, k3, v3, Wq, Wo)
    return out
